# baseline (speedup 1.0000x reference)
import functools

import jax
import jax.numpy as jnp
from jax import lax
from jax.experimental import pallas as pl
from jax.experimental.pallas import tpu as pltpu
from jax.experimental.pallas import tpu_sc as plsc

_D = 64
_NW = 32
_CHUNK = 1024


@jax.jit
def _gather_flat(embedding, idx):
    btot = idx.shape[0]
    b_per_w = btot // _NW
    n_chunks = b_per_w // _CHUNK

    mesh = plsc.VectorSubcoreMesh(core_axis_name="c", subcore_axis_name="s")

    @functools.partial(
        pl.kernel,
        mesh=mesh,
        out_type=jax.ShapeDtypeStruct((btot, _D), jnp.float32),
        scratch_types=[
            pltpu.VMEM((_CHUNK,), jnp.int32),
            pltpu.VMEM((_CHUNK, _D), jnp.float32),
            pltpu.SemaphoreType.DMA,
        ],
        compiler_params=pltpu.CompilerParams(use_tc_tiling_on_sc=False),
    )
    def k(table_hbm, idx_hbm, out_hbm, idx_v, rows_v, sem):
        wid = lax.axis_index("s") * 2 + lax.axis_index("c")
        base = wid * b_per_w

        def body(g, carry):
            off = base + g * _CHUNK
            pltpu.sync_copy(idx_hbm.at[pl.ds(off, _CHUNK)], idx_v)
            pltpu.async_copy(table_hbm.at[idx_v], rows_v, sem).wait()
            pltpu.sync_copy(rows_v, out_hbm.at[pl.ds(off, _CHUNK)])
            return carry

        lax.fori_loop(0, n_chunks, body, 0)

    return k(embedding, idx)


def kernel(x, embedding):
    b, h = x.shape
    idx = x.reshape(b * h).astype(jnp.int32)
    out = _gather_flat(embedding, idx)
    return out.reshape(b, h, _D)


# speedup vs baseline: 1.0033x; 1.0033x over previous
import functools

import jax
import jax.numpy as jnp
from jax import lax
from jax.experimental import pallas as pl
from jax.experimental.pallas import tpu as pltpu
from jax.experimental.pallas import tpu_sc as plsc

_D = 64
_NW = 32
_CHUNK = 640


@jax.jit
def _gather_flat(embedding, idx):
    btot = idx.shape[0]
    b_per_w = btot // _NW
    n_chunks = b_per_w // _CHUNK

    mesh = plsc.VectorSubcoreMesh(core_axis_name="c", subcore_axis_name="s")

    @functools.partial(
        pl.kernel,
        mesh=mesh,
        out_type=jax.ShapeDtypeStruct((btot, _D), jnp.float32),
        scratch_types=[
            pltpu.VMEM((b_per_w,), jnp.int32),
            pltpu.VMEM((_CHUNK, _D), jnp.float32),
            pltpu.VMEM((_CHUNK, _D), jnp.float32),
            pltpu.SemaphoreType.DMA,
            pltpu.SemaphoreType.DMA,
            pltpu.SemaphoreType.DMA,
            pltpu.SemaphoreType.DMA,
        ],
        compiler_params=pltpu.CompilerParams(use_tc_tiling_on_sc=False),
    )
    def k(table_hbm, idx_hbm, out_hbm, idx_all, rows0, rows1,
          gsem0, gsem1, wsem0, wsem1):
        wid = lax.axis_index("s") * 2 + lax.axis_index("c")
        base = wid * b_per_w
        rows = (rows0, rows1)
        gsem = (gsem0, gsem1)
        wsem = (wsem0, wsem1)

        pltpu.sync_copy(idx_hbm.at[pl.ds(base, b_per_w)], idx_all)

        def gather(c):
            return pltpu.async_copy(
                table_hbm.at[idx_all.at[pl.ds(c * _CHUNK, _CHUNK)]],
                rows[c % 2], gsem[c % 2])

        def writeback(c):
            return pltpu.async_copy(
                rows[c % 2], out_hbm.at[pl.ds(base + c * _CHUNK, _CHUNK)],
                wsem[c % 2])

        g_pending = gather(0)
        w_pending = [None, None]
        for c in range(n_chunks):
            s = c % 2
            g_pending.wait()
            if c + 1 < n_chunks:
                if w_pending[1 - s] is not None:
                    w_pending[1 - s].wait()
                g_pending = gather(c + 1)
            w_pending[s] = writeback(c)
        w_pending[(n_chunks - 2) % 2].wait()
        w_pending[(n_chunks - 1) % 2].wait()

    return k(embedding, idx)


def kernel(x, embedding):
    b, h = x.shape
    idx = x.reshape(b * h).astype(jnp.int32)
    out = _gather_flat(embedding, idx)
    return out.reshape(b, h, _D)


# speedup vs baseline: 1.0090x; 1.0056x over previous
import functools

import jax
import jax.numpy as jnp
from jax import lax
from jax.experimental import pallas as pl
from jax.experimental.pallas import tpu as pltpu
from jax.experimental.pallas import tpu_sc as plsc

_D = 64
_NW = 32


@jax.jit
def _gather_nn(embedding, x_t):
    h, b = x_t.shape
    bw = b // _NW

    mesh = plsc.VectorSubcoreMesh(core_axis_name="c", subcore_axis_name="s")

    @functools.partial(
        pl.kernel,
        mesh=mesh,
        out_type=jax.ShapeDtypeStruct((b, h, _D), jnp.float32),
        scratch_types=[
            pltpu.VMEM((h, bw), jnp.int32),
            pltpu.VMEM((bw, _D), jnp.float32),
            pltpu.VMEM((bw, _D), jnp.float32),
            pltpu.SemaphoreType.DMA,
            pltpu.SemaphoreType.DMA,
            pltpu.SemaphoreType.DMA,
            pltpu.SemaphoreType.DMA,
        ],
        compiler_params=pltpu.CompilerParams(use_tc_tiling_on_sc=False),
    )
    def k(table_hbm, xt_hbm, out_hbm, idx_t, rows0, rows1,
          gsem0, gsem1, wsem0, wsem1):
        wid = lax.axis_index("s") * 2 + lax.axis_index("c")
        base = wid * bw
        rows = (rows0, rows1)
        gsem = (gsem0, gsem1)
        wsem = (wsem0, wsem1)

        pltpu.sync_copy(xt_hbm.at[:, pl.ds(base, bw)], idx_t)

        def gather(c):
            return pltpu.async_copy(
                table_hbm.at[idx_t.at[c]], rows[c % 2], gsem[c % 2])

        def writeback(c):
            return pltpu.async_copy(
                rows[c % 2], out_hbm.at[pl.ds(base, bw), c], wsem[c % 2])

        g_pending = gather(0)
        w_pending = [None, None]
        for c in range(h):
            s = c % 2
            g_pending.wait()
            if c + 1 < h:
                if w_pending[1 - s] is not None:
                    w_pending[1 - s].wait()
                g_pending = gather(c + 1)
            w_pending[s] = writeback(c)
        w_pending[(h - 2) % 2].wait()
        w_pending[(h - 1) % 2].wait()

    return k(embedding, x_t)


def kernel(x, embedding):
    b, h = x.shape
    out = _gather_nn(embedding, jnp.swapaxes(x, 0, 1))
    return out


# speedup vs baseline: 1.0334x; 1.0242x over previous
import functools

import jax
import jax.numpy as jnp
from jax import lax
from jax.experimental import pallas as pl
from jax.experimental.pallas import tpu as pltpu
from jax.experimental.pallas import tpu_sc as plsc

_D = 64
_DP = 128
_NW = 32
_CB = 256


@jax.jit
def _gather_nn(emb_pad, x_t):
    h, b = x_t.shape
    bw = b // _NW
    nsub = bw // _CB
    n_chunks = h * nsub

    mesh = plsc.VectorSubcoreMesh(core_axis_name="c", subcore_axis_name="s")

    @functools.partial(
        pl.kernel,
        mesh=mesh,
        out_type=jax.ShapeDtypeStruct((b, h, _D), jnp.float32),
        scratch_types=[
            pltpu.VMEM((h, bw), jnp.int32),
            pltpu.VMEM((_CB, _DP), jnp.float32),
            pltpu.VMEM((_CB, _DP), jnp.float32),
            pltpu.SemaphoreType.DMA,
            pltpu.SemaphoreType.DMA,
            pltpu.SemaphoreType.DMA,
            pltpu.SemaphoreType.DMA,
        ],
        compiler_params=pltpu.CompilerParams(use_tc_tiling_on_sc=False),
    )
    def k(table_hbm, xt_hbm, out_hbm, idx_t, rows0, rows1,
          gsem0, gsem1, wsem0, wsem1):
        wid = lax.axis_index("s") * 2 + lax.axis_index("c")
        base = wid * bw
        rows = (rows0, rows1)
        gsem = (gsem0, gsem1)
        wsem = (wsem0, wsem1)

        pltpu.sync_copy(xt_hbm.at[:, pl.ds(base, bw)], idx_t)

        def gather(c):
            hh, half = c // nsub, c % nsub
            return pltpu.async_copy(
                table_hbm.at[idx_t.at[hh, pl.ds(half * _CB, _CB)]],
                rows[c % 2], gsem[c % 2])

        def writeback(c):
            hh, half = c // nsub, c % nsub
            return pltpu.async_copy(
                rows[c % 2].at[:, pl.ds(0, _D)],
                out_hbm.at[pl.ds(base + half * _CB, _CB), hh],
                wsem[c % 2])

        g_pending = gather(0)
        w_pending = [None, None]
        for c in range(n_chunks):
            s = c % 2
            g_pending.wait()
            if c + 1 < n_chunks:
                if w_pending[1 - s] is not None:
                    w_pending[1 - s].wait()
                g_pending = gather(c + 1)
            w_pending[s] = writeback(c)
        w_pending[(n_chunks - 2) % 2].wait()
        w_pending[(n_chunks - 1) % 2].wait()

    return k(emb_pad, x_t)


def kernel(x, embedding):
    emb_pad = jnp.pad(embedding, ((0, 0), (0, _DP - _D)))
    return _gather_nn(emb_pad, jnp.swapaxes(x, 0, 1))
